# baseline (speedup 1.0000x reference)
import functools

import jax
import jax.numpy as jnp
from jax import lax
from jax.experimental import pallas as pl
from jax.experimental.pallas import tpu as pltpu
from jax.experimental.pallas import tpu_sc as plsc

N = 10000
D = 128
E = 320000

NC = 2
NS = 16
NW = NC * NS
CHUNK = 120
CH = 84
E_PAD = NW * CH * CHUNK
N_PAD = 10112
RPS = N_PAD // NS
UNROLL = 6
NIB = 6
NRB = 3


def _sc_aggregate(x, src, dst, zeros):
    mesh = plsc.VectorSubcoreMesh(core_axis_name="c", subcore_axis_name="s")

    @functools.partial(
        pl.kernel,
        out_type=jax.ShapeDtypeStruct((NC, N_PAD, D), jnp.float32),
        mesh=mesh,
        scratch_types=[
            [pltpu.VMEM((CHUNK,), jnp.int32) for _ in range(NIB)],
            [pltpu.VMEM((CHUNK,), jnp.int32) for _ in range(NIB)],
            [pltpu.VMEM((CHUNK, D), jnp.float32) for _ in range(NRB)],
            pltpu.VMEM_SHARED((N_PAD, D), jnp.float32),
            [pltpu.SemaphoreType.DMA for _ in range(NIB)],
            [pltpu.SemaphoreType.DMA for _ in range(NRB)],
            [pltpu.SemaphoreType.DMA for _ in range(NRB)],
        ],
    )
    def agg_kernel(x_hbm, src_hbm, dst_hbm, z_hbm, out_hbm,
                   sidx, didx, rows, acc_sh, isem, gsem, ssem):
        c = lax.axis_index("c")
        s = lax.axis_index("s")
        wid = c * NS + s
        base = wid * (CH * CHUNK)

        def start_idx(i, r):
            off = base + i * CHUNK
            pltpu.async_copy(src_hbm.at[pl.ds(off, CHUNK)], sidx[r], isem[r])
            pltpu.async_copy(dst_hbm.at[pl.ds(off, CHUNK)], didx[r], isem[r])

        def wait_idx(r):
            pltpu.make_async_copy(
                src_hbm.at[pl.ds(0, CHUNK)], sidx[r], isem[r]).wait()
            pltpu.make_async_copy(
                dst_hbm.at[pl.ds(0, CHUNK)], didx[r], isem[r]).wait()

        def start_gather(r, b):
            pltpu.async_copy(x_hbm.at[sidx[r]], rows[b], gsem[b])

        def wait_gather(b):
            pltpu.make_async_copy(
                x_hbm.at[sidx[0]], rows[b], gsem[b]).wait()

        def start_scatter(r, b):
            pltpu.async_copy(rows[b], acc_sh.at[didx[r]], ssem[b], add=True)

        def wait_scatter(b):
            pltpu.make_async_copy(
                rows[b], acc_sh.at[didx[0]], ssem[b]).wait()

        for r in range(4):
            start_idx(r, r)
        pltpu.sync_copy(z_hbm.at[pl.ds(s * RPS, RPS)],
                        acc_sh.at[pl.ds(s * RPS, RPS)])
        wait_idx(0)
        start_gather(0, 0)
        wait_idx(1)
        start_gather(1, 1)
        plsc.subcore_barrier()

        def body(g, carry):
            for k in range(UNROLL):
                i = UNROLL * g + k
                b = k % NRB
                wait_gather(b)
                start_scatter(k, b)

                @pl.when(i > 0)
                def _():
                    wait_scatter((k + NRB - 1) % NRB)

                @pl.when(i + 4 < CH)
                def _():
                    start_idx(i + 4, (k + 4) % NIB)

                @pl.when(i + 2 < CH)
                def _():
                    wait_idx((k + 2) % NIB)
                    start_gather((k + 2) % NIB, (k + 2) % NRB)

            return carry

        lax.fori_loop(0, CH // UNROLL, body, 0)
        wait_scatter((CH - 1) % NRB)
        plsc.subcore_barrier()

        pltpu.sync_copy(acc_sh.at[pl.ds(s * RPS, RPS)],
                        out_hbm.at[c, pl.ds(s * RPS, RPS)])

    return agg_kernel(x, src, dst, zeros)


BLK = 1000


def _xw1_kernel(eps_ref, x_ref, w1_ref, b1_ref, a_ref):
    h = x_ref[...] * (1.0 + eps_ref[0])
    a_ref[...] = (jnp.dot(h, w1_ref[...], preferred_element_type=jnp.float32,
                          precision=lax.Precision.HIGHEST)
                  + b1_ref[...])


def _xw1(eps, x, W1, b1):
    grid = (N // BLK,)
    row_spec = pl.BlockSpec((BLK, D), lambda i: (i, 0))
    full_spec = pl.BlockSpec((D, D), lambda i: (0, 0))
    bias_spec = pl.BlockSpec((1, D), lambda i: (0, 0))
    return pl.pallas_call(
        _xw1_kernel,
        grid=grid,
        in_specs=[pl.BlockSpec(memory_space=pltpu.SMEM),
                  row_spec, full_spec, bias_spec],
        out_specs=row_spec,
        out_shape=jax.ShapeDtypeStruct((N, D), jnp.float32),
    )(eps, x, W1, b1)


def _mlp_kernel(a_ref, p_ref, w1_ref, w2_ref, b2_ref, o_ref):
    agg = p_ref[0, :, :] + p_ref[1, :, :]
    h = a_ref[...] + jnp.dot(agg, w1_ref[...],
                             preferred_element_type=jnp.float32,
                             precision=lax.Precision.HIGHEST)
    h = jnp.maximum(h, 0.0)
    o = jnp.dot(h, w2_ref[...], preferred_element_type=jnp.float32,
                precision=lax.Precision.HIGHEST)
    o_ref[...] = o + b2_ref[...]


def _mlp(a, partials, W1, W2, b2):
    grid = (N // BLK,)
    row_spec = pl.BlockSpec((BLK, D), lambda i: (i, 0))
    part_spec = pl.BlockSpec((NC, BLK, D), lambda i: (0, i, 0))
    full_spec = pl.BlockSpec((D, D), lambda i: (0, 0))
    bias_spec = pl.BlockSpec((1, D), lambda i: (0, 0))
    return pl.pallas_call(
        _mlp_kernel,
        grid=grid,
        in_specs=[row_spec, part_spec, full_spec, full_spec, bias_spec],
        out_specs=row_spec,
        out_shape=jax.ShapeDtypeStruct((N, D), jnp.float32),
    )(a, partials, W1, W2, b2)


def kernel(x, edge_index, eps, W1, b1, W2, b2):
    src = edge_index[0].astype(jnp.int32)
    dst = edge_index[1].astype(jnp.int32)
    pad = E_PAD - E
    pad_ids = jnp.arange(pad, dtype=jnp.int32)
    src_p = jnp.concatenate([src, pad_ids % N])
    dst_p = jnp.concatenate([dst, N + pad_ids % (N_PAD - N)])
    zeros = jnp.zeros((N_PAD, D), jnp.float32)

    partials = _sc_aggregate(x, src_p, dst_p, zeros)

    eps1 = jnp.reshape(eps, (1,)).astype(jnp.float32)
    a = _xw1(eps1, x, W1, jnp.reshape(b1, (1, D)))
    return _mlp(a, partials, W1, W2, jnp.reshape(b2, (1, D)))


# speedup vs baseline: 1.1705x; 1.1705x over previous
import functools

import jax
import jax.numpy as jnp
from jax import lax
from jax.experimental import pallas as pl
from jax.experimental.pallas import tpu as pltpu
from jax.experimental.pallas import tpu_sc as plsc

N = 10000
D = 128
E = 320000

NC = 2
NS = 16
NW = NC * NS
CHUNK = 120
CH = 84
E_PAD = NW * CH * CHUNK
N_PAD = 10112
RPS = N_PAD // NS
UNROLL = 6
NIB = 6
NRB = 3


def _sc_aggregate(x, src, dst, zeros):
    mesh = plsc.VectorSubcoreMesh(core_axis_name="c", subcore_axis_name="s")

    @functools.partial(
        pl.kernel,
        out_type=jax.ShapeDtypeStruct((NC, N_PAD, D), jnp.float32),
        mesh=mesh,
        scratch_types=[
            [pltpu.VMEM((CHUNK,), jnp.int32) for _ in range(NIB)],
            [pltpu.VMEM((CHUNK,), jnp.int32) for _ in range(NIB)],
            [pltpu.VMEM((CHUNK, D), jnp.float32) for _ in range(NRB)],
            pltpu.VMEM_SHARED((N_PAD, D), jnp.float32),
            [pltpu.SemaphoreType.DMA for _ in range(NIB)],
            [pltpu.SemaphoreType.DMA for _ in range(NRB)],
            [pltpu.SemaphoreType.DMA for _ in range(NRB)],
        ],
    )
    def agg_kernel(x_hbm, src_hbm, dst_hbm, z_hbm, out_hbm,
                   sidx, didx, rows, acc_sh, isem, gsem, ssem):
        c = lax.axis_index("c")
        s = lax.axis_index("s")
        wid = c * NS + s
        base = wid * (CH * CHUNK)

        def start_idx(i, r):
            off = base + i * CHUNK
            pltpu.async_copy(src_hbm.at[pl.ds(off, CHUNK)], sidx[r], isem[r])
            pltpu.async_copy(dst_hbm.at[pl.ds(off, CHUNK)], didx[r], isem[r])

        def wait_idx(r):
            pltpu.make_async_copy(
                src_hbm.at[pl.ds(0, CHUNK)], sidx[r], isem[r]).wait()
            pltpu.make_async_copy(
                dst_hbm.at[pl.ds(0, CHUNK)], didx[r], isem[r]).wait()

        def start_gather(r, b):
            pltpu.async_copy(x_hbm.at[sidx[r]], rows[b], gsem[b])

        def wait_gather(b):
            pltpu.make_async_copy(
                x_hbm.at[sidx[0]], rows[b], gsem[b]).wait()

        def start_scatter(r, b):
            pltpu.async_copy(rows[b], acc_sh.at[didx[r]], ssem[b], add=True)

        def wait_scatter(b):
            pltpu.make_async_copy(
                rows[b], acc_sh.at[didx[0]], ssem[b]).wait()

        for r in range(4):
            start_idx(r, r)
        pltpu.sync_copy(z_hbm.at[pl.ds(s * RPS, RPS)],
                        acc_sh.at[pl.ds(s * RPS, RPS)])
        wait_idx(0)
        start_gather(0, 0)
        wait_idx(1)
        start_gather(1, 1)
        plsc.subcore_barrier()

        def body(g, carry):
            for k in range(UNROLL):
                i = UNROLL * g + k
                b = k % NRB
                wait_gather(b)
                start_scatter(k, b)

                @pl.when(i > 0)
                def _():
                    wait_scatter((k + NRB - 1) % NRB)

                @pl.when(i + 4 < CH)
                def _():
                    start_idx(i + 4, (k + 4) % NIB)

                @pl.when(i + 2 < CH)
                def _():
                    wait_idx((k + 2) % NIB)
                    start_gather((k + 2) % NIB, (k + 2) % NRB)

            return carry

        lax.fori_loop(0, CH // UNROLL, body, 0)
        wait_scatter((CH - 1) % NRB)
        plsc.subcore_barrier()

        pltpu.sync_copy(acc_sh.at[pl.ds(s * RPS, RPS)],
                        out_hbm.at[c, pl.ds(s * RPS, RPS)])

    return agg_kernel(x, src, dst, zeros)


BLK = 2000


def _xw1_kernel(eps_ref, x_ref, w1_ref, b1_ref, a_ref):
    h = x_ref[...] * (1.0 + eps_ref[0])
    a_ref[...] = (jnp.dot(h, w1_ref[...], preferred_element_type=jnp.float32)
                  + b1_ref[...])


def _xw1(eps, x, W1, b1):
    grid = (N // BLK,)
    row_spec = pl.BlockSpec((BLK, D), lambda i: (i, 0))
    full_spec = pl.BlockSpec((D, D), lambda i: (0, 0))
    bias_spec = pl.BlockSpec((1, D), lambda i: (0, 0))
    return pl.pallas_call(
        _xw1_kernel,
        grid=grid,
        in_specs=[pl.BlockSpec(memory_space=pltpu.SMEM),
                  row_spec, full_spec, bias_spec],
        out_specs=row_spec,
        out_shape=jax.ShapeDtypeStruct((N, D), jnp.float32),
    )(eps, x, W1, b1)


def _mlp_kernel(a_ref, p_ref, w1_ref, w2_ref, b2_ref, o_ref):
    agg = p_ref[0, :, :] + p_ref[1, :, :]
    h = a_ref[...] + jnp.dot(agg, w1_ref[...],
                             preferred_element_type=jnp.float32)
    h = jnp.maximum(h, 0.0)
    o = jnp.dot(h, w2_ref[...], preferred_element_type=jnp.float32)
    o_ref[...] = o + b2_ref[...]


def _mlp(a, partials, W1, W2, b2):
    grid = (N // BLK,)
    row_spec = pl.BlockSpec((BLK, D), lambda i: (i, 0))
    part_spec = pl.BlockSpec((NC, BLK, D), lambda i: (0, i, 0))
    full_spec = pl.BlockSpec((D, D), lambda i: (0, 0))
    bias_spec = pl.BlockSpec((1, D), lambda i: (0, 0))
    return pl.pallas_call(
        _mlp_kernel,
        grid=grid,
        in_specs=[row_spec, part_spec, full_spec, full_spec, bias_spec],
        out_specs=row_spec,
        out_shape=jax.ShapeDtypeStruct((N, D), jnp.float32),
    )(a, partials, W1, W2, b2)


def kernel(x, edge_index, eps, W1, b1, W2, b2):
    src = edge_index[0].astype(jnp.int32)
    dst = edge_index[1].astype(jnp.int32)
    pad = E_PAD - E
    pad_ids = jnp.arange(pad, dtype=jnp.int32)
    src_p = jnp.concatenate([src, pad_ids % N])
    dst_p = jnp.concatenate([dst, N + pad_ids % (N_PAD - N)])
    zeros = jnp.zeros((N_PAD, D), jnp.float32)

    partials = _sc_aggregate(x, src_p, dst_p, zeros)

    eps1 = jnp.reshape(eps, (1,)).astype(jnp.float32)
    a = _xw1(eps1, x, W1, jnp.reshape(b1, (1, D)))
    return _mlp(a, partials, W1, W2, jnp.reshape(b2, (1, D)))
